# baseline (device time: 42049 ns/iter reference)
import jax
import jax.numpy as jnp
from jax import lax
from jax.experimental import pallas as pl
from jax.experimental.pallas import tpu as pltpu

N_DEV = 8
HQ = 8
DH = 128
SQ = 256
D = 1024
SCALE = 0.08838834764831843
N_ROUNDS = 3
NCH = 4
HCH = HQ // NCH
NSLOT = 4
C3 = NCH - 1


def kernel(x, Wq, Wo, K_ext, V_ext):
    skv = K_ext.shape[1]
    x2 = x.reshape(SQ, D)
    k3 = K_ext.reshape(skv, HQ, DH)
    v3 = V_ext.reshape(skv, HQ, DH)

    def body(x_ref, wq_ref, wo_ref, k_ref, v_ref, out_ref,
             kbuf, vbuf, wobuf, outc, statc, a2ao, a2as,
             copy_sems, wo_sem, send_sems, recv_sems, a2a_send, a2a_recv):
        my = lax.axis_index("i")

        barrier = pltpu.get_barrier_semaphore()
        for d in range(1, N_DEV):
            pl.semaphore_signal(
                barrier, inc=1,
                device_id=(my ^ d,),
                device_id_type=pl.DeviceIdType.MESH,
            )

        def head_dma(h):
            slot = h % NSLOT
            return (
                pltpu.make_async_copy(
                    k_ref.at[:, h, :], kbuf.at[slot], copy_sems.at[slot, 0]
                ),
                pltpu.make_async_copy(
                    v_ref.at[:, h, :], vbuf.at[slot], copy_sems.at[slot, 1]
                ),
            )

        def round_rdmas(r, c):
            partner = my ^ (1 << r)
            hs = HCH * c
            o = pltpu.make_async_remote_copy(
                src_ref=outc.at[2 * r, hs:hs + HCH],
                dst_ref=outc.at[2 * r + 1, hs:hs + HCH],
                send_sem=send_sems.at[r, c, 0],
                recv_sem=recv_sems.at[r, c, 0],
                device_id=(partner,),
                device_id_type=pl.DeviceIdType.MESH,
            )
            rs = 4 * c
            s = pltpu.make_async_remote_copy(
                src_ref=statc.at[2 * r, rs:rs + 4],
                dst_ref=statc.at[2 * r + 1, rs:rs + 4],
                send_sem=send_sems.at[r, c, 1],
                recv_sem=recv_sems.at[r, c, 1],
                device_id=(partner,),
                device_id_type=pl.DeviceIdType.MESH,
            )
            return o, s

        rdmas = {(r, c): round_rdmas(r, c)
                 for r in range(N_ROUNDS) for c in range(NCH - 1)}

        def a2a_rdmas(d):
            partner = my ^ d
            o = pltpu.make_async_remote_copy(
                src_ref=outc.at[0, HCH * C3:HCH * C3 + HCH],
                dst_ref=a2ao.at[d - 1],
                send_sem=a2a_send.at[d - 1, 0],
                recv_sem=a2a_recv.at[d - 1, 0],
                device_id=(partner,),
                device_id_type=pl.DeviceIdType.MESH,
            )
            s = pltpu.make_async_remote_copy(
                src_ref=statc.at[0, 4 * C3:4 * C3 + 4],
                dst_ref=a2as.at[d - 1],
                send_sem=a2a_send.at[d - 1, 1],
                recv_sem=a2a_recv.at[d - 1, 1],
                device_id=(partner,),
                device_id_type=pl.DeviceIdType.MESH,
            )
            return o, s

        a2a = {d: a2a_rdmas(d) for d in range(1, N_DEV)}

        def merge(r, c):
            rs = 4 * c
            hs = HCH * c
            m1 = statc[2 * r, rs:rs + 2, :]
            l1 = statc[2 * r, rs + 2:rs + 4, :]
            m2 = statc[2 * r + 1, rs:rs + 2, :]
            l2 = statc[2 * r + 1, rs + 2:rs + 4, :]
            mm = jnp.maximum(m1, m2)
            w1 = jnp.exp(m1 - mm)
            w2 = jnp.exp(m2 - mm)
            lm = w1 * l1 + w2 * l2
            om = (w1[:, None, :] * outc[2 * r, hs:hs + HCH].astype(jnp.float32)
                  + w2[:, None, :] * outc[2 * r + 1, hs:hs + HCH].astype(jnp.float32))
            return om, mm, lm

        accs = [None] * NCH

        def wo_chunk(om, lm, c):
            attn = om / lm[:, None, :]
            acc = None
            for j in range(HCH):
                h = HCH * c + j
                ah = attn[j].astype(jnp.bfloat16)
                woh = wobuf[h * DH:(h + 1) * DH, :].astype(jnp.bfloat16)
                d = lax.dot_general(ah, woh, (((0,), (0,)), ((), ())),
                                    preferred_element_type=jnp.float32)
                acc = d if acc is None else acc + d
            accs[c] = acc

        def process(r, c):
            ro, rst = rdmas[(r, c)]
            ro.wait_recv(); rst.wait_recv()
            om, mm, lm = merge(r, c)
            if r < N_ROUNDS - 1:
                rs = 4 * c
                hs = HCH * c
                outc[2 * (r + 1), hs:hs + HCH] = om.astype(jnp.bfloat16)
                statc[2 * (r + 1), rs:rs + 2, :] = mm
                statc[2 * (r + 1), rs + 2:rs + 4, :] = lm
                no, ns = rdmas[(r + 1, c)]
                no.start(); ns.start()
            else:
                wo_chunk(om, lm, c)

        def process_c3():
            for d in range(1, N_DEV):
                o, s = a2a[d]
                o.wait_recv(); s.wait_recv()
            ms = [statc[0, 4 * C3:4 * C3 + 2, :]]
            ls = [statc[0, 4 * C3 + 2:4 * C3 + 4, :]]
            os_ = [outc[0, HCH * C3:HCH * C3 + HCH].astype(jnp.float32)]
            for d in range(1, N_DEV):
                ms.append(a2as[d - 1, 0:2, :])
                ls.append(a2as[d - 1, 2:4, :])
                os_.append(a2ao[d - 1].astype(jnp.float32))
            mm = ms[0]
            for m_ in ms[1:]:
                mm = jnp.maximum(mm, m_)
            lm = None
            om = None
            for m_, l_, o_ in zip(ms, ls, os_):
                w = jnp.exp(m_ - mm)
                lw = w * l_
                ow = w[:, None, :] * o_
                lm = lw if lm is None else lm + lw
                om = ow if om is None else om + ow
            wo_chunk(om, lm, C3)

        for h0 in range(3):
            kd, vd = head_dma(h0)
            kd.start(); vd.start()
        wo_copy = pltpu.make_async_copy(wo_ref, wobuf, wo_sem)
        wo_copy.start()

        xq = x_ref[:, :].astype(jnp.bfloat16)
        wq = wq_ref[:, :].astype(jnp.bfloat16)
        q = lax.dot_general(xq, wq, (((1,), (0,)), ((), ())),
                            preferred_element_type=jnp.float32)
        qs = (q * SCALE).astype(jnp.bfloat16)

        pl.semaphore_wait(barrier, N_DEV - 1)

        def qk(h):
            kd, vd = head_dma(h)
            kd.wait(); vd.wait()
            kh = kbuf[h % NSLOT].astype(jnp.bfloat16)
            qh = qs[:, h * DH:(h + 1) * DH]
            return lax.dot_general(kh, qh, (((1,), (1,)), ((), ())),
                                   preferred_element_type=jnp.float32)

        in_loop = {4: (0, 0), 6: (0, 1), 7: (1, 0)}
        post = [(0, 2), (1, 1), (2, 0), (1, 2), "c3", (2, 1), (2, 2)]

        st_cur = qk(0)
        for h in range(HQ):
            st_next = qk(h + 1) if h + 1 < HQ else None
            c, i = divmod(h, HCH)
            mt = jnp.max(st_cur, axis=0, keepdims=True)
            pt = jnp.exp((st_cur - mt).astype(jnp.bfloat16))
            lt = jnp.sum(pt, axis=0, keepdims=True, dtype=jnp.float32)
            vh = vbuf[h % NSLOT].astype(jnp.bfloat16)
            ot = lax.dot_general(vh, pt, (((0,), (0,)), ((), ())),
                                 preferred_element_type=jnp.float32)
            if h + 3 < HQ:
                nkd, nvd = head_dma(h + 3)
                nkd.start(); nvd.start()
            outc[0, h] = ot.astype(jnp.bfloat16)
            rs = 4 * c
            statc[0, rs + i:rs + i + 1, :] = mt
            statc[0, rs + 2 + i:rs + 2 + i + 1, :] = lt
            if i == HCH - 1:
                if c < C3:
                    ro, rst = rdmas[(0, c)]
                    ro.start(); rst.start()
                else:
                    for d in range(1, N_DEV):
                        o, s = a2a[d]
                        o.start(); s.start()
            if h in in_loop:
                process(*in_loop[h])
            st_cur = st_next

        wo_copy.wait()
        for ev in post:
            if ev == "c3":
                process_c3()
            else:
                process(*ev)

        out_ref[:, :] = (accs[0] + accs[1]) + (accs[2] + accs[3])

        for r in range(N_ROUNDS):
            for c in range(NCH - 1):
                ro, rst = rdmas[(r, c)]
                ro.wait_send(); rst.wait_send()
        for d in range(1, N_DEV):
            o, s = a2a[d]
            o.wait_send(); s.wait_send()

    out = pl.pallas_call(
        body,
        out_shape=jax.ShapeDtypeStruct((SQ, D), jnp.float32),
        in_specs=[pl.BlockSpec(memory_space=pltpu.VMEM)] * 2
        + [pl.BlockSpec(memory_space=pltpu.MemorySpace.HBM)] * 3,
        out_specs=pl.BlockSpec(memory_space=pltpu.VMEM),
        scratch_shapes=[
            pltpu.VMEM((NSLOT, 4096, DH), jnp.float32),
            pltpu.VMEM((NSLOT, 4096, DH), jnp.float32),
            pltpu.VMEM((D, D), jnp.float32),
            pltpu.VMEM((2 * N_ROUNDS, HQ, DH, SQ), jnp.bfloat16),
            pltpu.VMEM((2 * N_ROUNDS, 2 * HQ, SQ), jnp.float32),
            pltpu.VMEM((N_DEV - 1, HCH, DH, SQ), jnp.bfloat16),
            pltpu.VMEM((N_DEV - 1, 4, SQ), jnp.float32),
            pltpu.SemaphoreType.DMA((NSLOT, 2)),
            pltpu.SemaphoreType.DMA,
            pltpu.SemaphoreType.DMA((N_ROUNDS, NCH - 1, 2)),
            pltpu.SemaphoreType.DMA((N_ROUNDS, NCH - 1, 2)),
            pltpu.SemaphoreType.DMA((N_DEV - 1, 2)),
            pltpu.SemaphoreType.DMA((N_DEV - 1, 2)),
        ],
        compiler_params=pltpu.CompilerParams(
            collective_id=0,
            vmem_limit_bytes=100 * 1024 * 1024,
        ),
    )(x2, Wq, Wo, k3, v3)
    return out.reshape(1, SQ, D)


# device time: 38547 ns/iter; 1.0909x vs baseline; 1.0909x over previous
import jax
import jax.numpy as jnp
from jax import lax
from jax.experimental import pallas as pl
from jax.experimental.pallas import tpu as pltpu

N_DEV = 8
HQ = 8
DH = 128
SQ = 256
D = 1024
SCALE = 0.08838834764831843
N_ROUNDS = 3
NCH = 4
HCH = HQ // NCH
NSLOT = 4


def kernel(x, Wq, Wo, K_ext, V_ext):
    skv = K_ext.shape[1]
    x2 = x.reshape(SQ, D)
    k3 = K_ext.reshape(skv, HQ, DH)
    v3 = V_ext.reshape(skv, HQ, DH)

    def body(x_ref, wq_ref, wo_ref, k_ref, v_ref, out_ref,
             kbuf, vbuf, wobuf, outc, statc,
             copy_sems, wo_sem, send_sems, recv_sems):
        my = lax.axis_index("i")

        barrier = pltpu.get_barrier_semaphore()
        for r in range(N_ROUNDS):
            pl.semaphore_signal(
                barrier, inc=1,
                device_id=(my ^ (1 << r),),
                device_id_type=pl.DeviceIdType.MESH,
            )

        def head_dma(h):
            slot = h % NSLOT
            return (
                pltpu.make_async_copy(
                    k_ref.at[:, h, :], kbuf.at[slot], copy_sems.at[slot, 0]
                ),
                pltpu.make_async_copy(
                    v_ref.at[:, h, :], vbuf.at[slot], copy_sems.at[slot, 1]
                ),
            )

        def round_rdmas(r, c):
            partner = my ^ (1 << r)
            hs = HCH * c
            o = pltpu.make_async_remote_copy(
                src_ref=outc.at[2 * r, hs:hs + HCH],
                dst_ref=outc.at[2 * r + 1, hs:hs + HCH],
                send_sem=send_sems.at[r, c, 0],
                recv_sem=recv_sems.at[r, c, 0],
                device_id=(partner,),
                device_id_type=pl.DeviceIdType.MESH,
            )
            rs = 4 * c
            s = pltpu.make_async_remote_copy(
                src_ref=statc.at[2 * r, rs:rs + 4],
                dst_ref=statc.at[2 * r + 1, rs:rs + 4],
                send_sem=send_sems.at[r, c, 1],
                recv_sem=recv_sems.at[r, c, 1],
                device_id=(partner,),
                device_id_type=pl.DeviceIdType.MESH,
            )
            return o, s

        rdmas = {(r, c): round_rdmas(r, c)
                 for r in range(N_ROUNDS) for c in range(NCH)}


        def merge(r, c):
            rs = 4 * c
            hs = HCH * c
            m1 = statc[2 * r, rs:rs + 2, :]
            l1 = statc[2 * r, rs + 2:rs + 4, :]
            m2 = statc[2 * r + 1, rs:rs + 2, :]
            l2 = statc[2 * r + 1, rs + 2:rs + 4, :]
            mm = jnp.maximum(m1, m2)
            w1 = jnp.exp2(m1 - mm)
            w2 = jnp.exp2(m2 - mm)
            lm = w1 * l1 + w2 * l2
            om = (w1[:, None, :] * outc[2 * r, hs:hs + HCH].astype(jnp.float32)
                  + w2[:, None, :] * outc[2 * r + 1, hs:hs + HCH].astype(jnp.float32))
            return om, mm, lm

        accs = [None] * NCH

        def wo_chunk(om, lm, c):
            attn = om / lm[:, None, :]
            acc = None
            for j in range(HCH):
                h = HCH * c + j
                ah = attn[j].astype(jnp.bfloat16)
                woh = wobuf[h * DH:(h + 1) * DH, :].astype(jnp.bfloat16)
                d = lax.dot_general(ah, woh, (((0,), (0,)), ((), ())),
                                    preferred_element_type=jnp.float32)
                acc = d if acc is None else acc + d
            accs[c] = acc

        def process(r, c):
            ro, rst = rdmas[(r, c)]
            ro.wait_recv(); rst.wait_recv()
            om, mm, lm = merge(r, c)
            if r < N_ROUNDS - 1:
                rs = 4 * c
                hs = HCH * c
                outc[2 * (r + 1), hs:hs + HCH] = om.astype(jnp.bfloat16)
                statc[2 * (r + 1), rs:rs + 2, :] = mm
                statc[2 * (r + 1), rs + 2:rs + 4, :] = lm
                no, ns = rdmas[(r + 1, c)]
                no.start(); ns.start()
            else:
                wo_chunk(om, lm, c)


        for h0 in range(3):
            kd, vd = head_dma(h0)
            kd.start(); vd.start()
        wo_copy = pltpu.make_async_copy(wo_ref, wobuf, wo_sem)
        wo_copy.start()

        xq = x_ref[:, :].astype(jnp.bfloat16)
        wq = wq_ref[:, :].astype(jnp.bfloat16)
        q = lax.dot_general(xq, wq, (((1,), (0,)), ((), ())),
                            preferred_element_type=jnp.float32)
        qs = (q * (SCALE * 1.4426950408889634)).astype(jnp.bfloat16)

        pl.semaphore_wait(barrier, N_ROUNDS)

        def qk(h):
            kd, vd = head_dma(h)
            kd.wait(); vd.wait()
            kh = kbuf[h % NSLOT].astype(jnp.bfloat16)
            qh = qs[:, h * DH:(h + 1) * DH]
            return lax.dot_general(kh, qh, (((1,), (1,)), ((), ())),
                                   preferred_element_type=jnp.float32)

        in_loop = {4: (0, 0), 6: (0, 1), 7: (1, 0)}
        post = [(0, 2), (1, 1), (2, 0), (0, 3), (1, 2),
                (2, 1), (1, 3), (2, 2), (2, 3)]

        st_cur = qk(0)
        for h in range(HQ):
            st_next = qk(h + 1) if h + 1 < HQ else None
            c, i = divmod(h, HCH)
            mt = jnp.max(st_cur, axis=0, keepdims=True)
            pt = jnp.exp2((st_cur - mt).astype(jnp.bfloat16))
            lt = jnp.sum(pt, axis=0, keepdims=True, dtype=jnp.float32)
            vh = vbuf[h % NSLOT].astype(jnp.bfloat16)
            ot = lax.dot_general(vh, pt, (((0,), (0,)), ((), ())),
                                 preferred_element_type=jnp.float32)
            if h + 3 < HQ:
                nkd, nvd = head_dma(h + 3)
                nkd.start(); nvd.start()
            outc[0, h] = ot.astype(jnp.bfloat16)
            rs = 4 * c
            statc[0, rs + i:rs + i + 1, :] = mt
            statc[0, rs + 2 + i:rs + 2 + i + 1, :] = lt
            if i == HCH - 1:
                ro, rst = rdmas[(0, c)]
                ro.start(); rst.start()
            if h in in_loop:
                process(*in_loop[h])
            st_cur = st_next

        wo_copy.wait()
        for ev in post:
            process(*ev)

        out_ref[:, :] = (accs[0] + accs[1]) + (accs[2] + accs[3])

        for r in range(N_ROUNDS):
            for c in range(NCH):
                ro, rst = rdmas[(r, c)]
                ro.wait_send(); rst.wait_send()

    out = pl.pallas_call(
        body,
        out_shape=jax.ShapeDtypeStruct((SQ, D), jnp.float32),
        in_specs=[pl.BlockSpec(memory_space=pltpu.VMEM)] * 2
        + [pl.BlockSpec(memory_space=pltpu.MemorySpace.HBM)] * 3,
        out_specs=pl.BlockSpec(memory_space=pltpu.VMEM),
        scratch_shapes=[
            pltpu.VMEM((NSLOT, 4096, DH), jnp.float32),
            pltpu.VMEM((NSLOT, 4096, DH), jnp.float32),
            pltpu.VMEM((D, D), jnp.float32),
            pltpu.VMEM((2 * N_ROUNDS, HQ, DH, SQ), jnp.bfloat16),
            pltpu.VMEM((2 * N_ROUNDS, 2 * HQ, SQ), jnp.float32),
            pltpu.SemaphoreType.DMA((NSLOT, 2)),
            pltpu.SemaphoreType.DMA,
            pltpu.SemaphoreType.DMA((N_ROUNDS, NCH, 2)),
            pltpu.SemaphoreType.DMA((N_ROUNDS, NCH, 2)),
        ],
        compiler_params=pltpu.CompilerParams(
            collective_id=0,
            vmem_limit_bytes=100 * 1024 * 1024,
        ),
    )(x2, Wq, Wo, k3, v3)
    return out.reshape(1, SQ, D)
